# Initial kernel scaffold; baseline (speedup 1.0000x reference)
import jax
import jax.numpy as jnp
from jax.experimental import pallas as pl


def kernel(src, dst, timestamps, edge_feats, memory, last_update_t, time_w, time_b, W_ih, W_hh, b_ih, b_hh):
    raise NotImplementedError("write your pallas kernel here")


# speedup vs baseline: 1.7626x; 1.7626x over previous
import functools

import jax
import jax.numpy as jnp
from jax import lax
from jax.experimental import pallas as pl
from jax.experimental.pallas import tpu as pltpu
from jax.experimental.pallas import tpu_sc as plsc

NUM_NODES = 100000
MEM_DIM = 128
EDGE_DIM = 16
TEMP_DIM = 128
B = 16384

NC = 2
NS = 16
NW = NC * NS
EV_PER_W = B // NW
IDX_CH = 128
CH_PER_W = EV_PER_W // IDX_CH

CUT = 20
ROWS_A = 3128
ROWS_B = 3120
CP_A = [136] * 23
CP_B = [136] * 22 + [128]
SB_PAD = 3136
SB_G = SB_PAD // 16
EV_G = B // 16
LIST_PAD = 3264

_mesh = plsc.VectorSubcoreMesh(
    core_axis_name="c", subcore_axis_name="s", num_cores=NC, num_subcores=NS)


def _wid():
  return lax.axis_index("s") * NC + lax.axis_index("c")


@functools.partial(
    pl.kernel,
    out_type=[
        jax.ShapeDtypeStruct((B, MEM_DIM), jnp.float32),
        jax.ShapeDtypeStruct((B, MEM_DIM), jnp.float32),
        jax.ShapeDtypeStruct((B,), jnp.float32),
    ],
    mesh=_mesh,
    scratch_types=[
        pltpu.VMEM((EV_PER_W,), jnp.int32),
        pltpu.VMEM((EV_PER_W,), jnp.int32),
        pltpu.VMEM((IDX_CH, MEM_DIM), jnp.float32),
        pltpu.VMEM((IDX_CH, MEM_DIM), jnp.float32),
        pltpu.VMEM((IDX_CH,), jnp.float32),
        pltpu.SemaphoreType.DMA,
        pltpu.SemaphoreType.DMA,
        pltpu.SemaphoreType.DMA,
    ],
    compiler_params=pltpu.CompilerParams(needs_layout_passes=False),
)
def _k1_gather(mem_hbm, lut_hbm, src_hbm, dst_hbm,
               msrc_hbm, mdst_hbm, lutg_hbm,
               isrc, idst, rowsA, rowsB, lutv, semA, semB, semL):
  wid = _wid()
  ebase = wid * EV_PER_W
  pltpu.sync_copy(src_hbm.at[pl.ds(ebase, EV_PER_W)], isrc)
  pltpu.sync_copy(dst_hbm.at[pl.ds(ebase, EV_PER_W)], idst)

  rows = (rowsA, rowsB)
  sems = (semA, semB)
  descs = []
  for j in range(2 * CH_PER_W):
    tbl = isrc if j < CH_PER_W else idst
    jj = j % CH_PER_W
    descs.append(pltpu.async_copy(
        mem_hbm.at[tbl.at[pl.ds(jj * IDX_CH, IDX_CH)]], rows[j % 2],
        sems[j % 2]))
    if j >= 1:
      k = j - 1
      descs[k].wait()
      out = msrc_hbm if k < CH_PER_W else mdst_hbm
      pltpu.sync_copy(rows[k % 2],
                      out.at[pl.ds(ebase + (k % CH_PER_W) * IDX_CH, IDX_CH)])
  k = 2 * CH_PER_W - 1
  descs[k].wait()
  pltpu.sync_copy(rowsB if k % 2 else rowsA,
                  mdst_hbm.at[pl.ds(ebase + (k % CH_PER_W) * IDX_CH, IDX_CH)])

  for j in range(CH_PER_W):
    pltpu.async_copy(
        lut_hbm.at[idst.at[pl.ds(j * IDX_CH, IDX_CH)]], lutv, semL).wait()
    pltpu.sync_copy(lutv, lutg_hbm.at[pl.ds(ebase + j * IDX_CH, IDX_CH)])


BM = 1024


def _k2_body(ms_ref, md_ref, ef_ref, ts_ref, lu_ref, tw_ref, tb_ref,
             ws_ref, wd_ref, we_ref, wt_ref, whh_ref, bih_ref, bhh_ref,
             out_ref):
  md = md_ref[...]
  dt = ts_ref[...] - lu_ref[...]
  tf = jnp.cos(dt * tw_ref[...] + tb_ref[...])
  f32 = jnp.float32
  gi = (jnp.dot(ms_ref[...], ws_ref[...], preferred_element_type=f32)
        + jnp.dot(md, wd_ref[...], preferred_element_type=f32)
        + jnp.dot(ef_ref[...], we_ref[...], preferred_element_type=f32)
        + jnp.dot(tf, wt_ref[...], preferred_element_type=f32)
        + bih_ref[...])
  gh = jnp.dot(md, whh_ref[...], preferred_element_type=f32) + bhh_ref[...]
  r = jax.nn.sigmoid(gi[:, :MEM_DIM] + gh[:, :MEM_DIM])
  z = jax.nn.sigmoid(gi[:, MEM_DIM:2 * MEM_DIM] + gh[:, MEM_DIM:2 * MEM_DIM])
  n = jnp.tanh(gi[:, 2 * MEM_DIM:] + r * gh[:, 2 * MEM_DIM:])
  out_ref[...] = (1.0 - z) * n + z * md


def _k2_call(msrc, mdst, ef, ts, lu, tw, tb, ws, wd, we, wt, whh, bih, bhh):
  g = B // BM
  row_spec = lambda w: pl.BlockSpec((BM, w), lambda i: (i, 0))
  full_spec = lambda a, b: pl.BlockSpec((a, b), lambda i: (0, 0))
  return pl.pallas_call(
      _k2_body,
      grid=(g,),
      in_specs=[
          row_spec(MEM_DIM), row_spec(MEM_DIM), row_spec(EDGE_DIM),
          row_spec(1), row_spec(1),
          full_spec(1, TEMP_DIM), full_spec(1, TEMP_DIM),
          full_spec(MEM_DIM, 3 * MEM_DIM), full_spec(MEM_DIM, 3 * MEM_DIM),
          full_spec(EDGE_DIM, 3 * MEM_DIM), full_spec(TEMP_DIM, 3 * MEM_DIM),
          full_spec(MEM_DIM, 3 * MEM_DIM),
          full_spec(1, 3 * MEM_DIM), full_spec(1, 3 * MEM_DIM),
      ],
      out_specs=row_spec(MEM_DIM),
      out_shape=jax.ShapeDtypeStruct((B, MEM_DIM), jnp.float32),
      compiler_params=pltpu.CompilerParams(
          dimension_semantics=("arbitrary",)),
  )(msrc, mdst, ef, ts, lu, tw, tb, ws, wd, we, wt, whh, bih, bhh)


@functools.partial(
    pl.kernel,
    out_type=jax.ShapeDtypeStruct((NUM_NODES, MEM_DIM), jnp.float32),
    mesh=_mesh,
    scratch_types=[
        pltpu.VMEM((136, MEM_DIM), jnp.float32),
        pltpu.VMEM((136, MEM_DIM), jnp.float32),
        pltpu.VMEM((B,), jnp.int32),
        pltpu.VMEM((SB_PAD,), jnp.int32),
        pltpu.VMEM((LIST_PAD,), jnp.int32),
        pltpu.VMEM((LIST_PAD,), jnp.int32),
        pltpu.VMEM((IDX_CH,), jnp.int32),
        pltpu.VMEM((IDX_CH,), jnp.int32),
        pltpu.VMEM((IDX_CH, MEM_DIM), jnp.float32),
        pltpu.VMEM((32,), jnp.int32),
        pltpu.SemaphoreType.DMA,
        pltpu.SemaphoreType.DMA,
        pltpu.SemaphoreType.DMA,
        pltpu.SemaphoreType.DMA,
        pltpu.SemaphoreType.DMA,
        pltpu.SemaphoreType.DMA,
    ],
    compiler_params=pltpu.CompilerParams(needs_layout_passes=False),
)
def _k3_scatter(mem_hbm, dst_hbm, hnew_hbm, out_hbm,
                cpA, cpB, dstv, sb, nodelist, winlist, gidx, nidx, rowbuf,
                nbuf, siA, siB, soA, soB, semG, semS):
  wid = _wid()
  row0 = wid * ROWS_B + 8 * jnp.minimum(wid, CUT)
  rows_w = jnp.where(wid < CUT, ROWS_A, ROWS_B)
  iota = lax.iota(jnp.int32, 16)

  dst_stage = pltpu.async_copy(dst_hbm, dstv, semG)

  def _init(g, _):
    sb[pl.ds(g * 16, 16)] = jnp.full((16,), -1, jnp.int32)
    return 0
  lax.fori_loop(0, SB_G, _init, 0)
  nbuf[pl.ds(16, 16)] = jnp.full((16,), 0x7FFFFFFF, jnp.int32)
  dst_stage.wait()

  def _dedup(g, _):
    d16 = dstv[pl.ds(g * 16, 16)]
    ids = g * 16 + iota
    rel = d16 - row0
    m = (rel >= 0) & (rel < rows_w)
    comp = jnp.where(m, rel * B + ids, jnp.int32(0x7FFFFFFF))
    sk, sv = plsc.sort_key_val(comp, ids)
    kidx = lax.shift_right_arithmetic(sk, 14)
    nbuf[pl.ds(0, 16)] = sk
    nxt = lax.shift_right_arithmetic(nbuf[pl.ds(1, 16)], 14)
    win = (kidx != nxt) & (sk != jnp.int32(0x7FFFFFFF))
    plsc.store_scatter(sb, [jnp.minimum(kidx, SB_PAD - 1)], sv, mask=win)
    return 0

  def _copy_loop(chunk_sizes):
    n = len(chunk_sizes)
    offs = [sum(chunk_sizes[:i]) for i in range(n)]
    bufs = (cpA, cpB)
    sin = (siA, siB)
    sout = (soA, soB)
    def cp_in(c):
      return pltpu.async_copy(
          mem_hbm.at[pl.ds(row0 + offs[c], chunk_sizes[c])],
          bufs[c % 2].at[pl.ds(0, chunk_sizes[c])], sin[c % 2])
    def cp_out(c):
      return pltpu.async_copy(
          bufs[c % 2].at[pl.ds(0, chunk_sizes[c])],
          out_hbm.at[pl.ds(row0 + offs[c], chunk_sizes[c])], sout[c % 2])
    gpc = (EV_G + n - 1) // n
    d_in = {0: cp_in(0)}
    d_out = {}
    for c in range(n):
      d_in[c].wait()
      d_out[c] = cp_out(c)
      if c + 1 < n:
        if c >= 1:
          d_out[c - 1].wait()
        d_in[c + 1] = cp_in(c + 1)
      lax.fori_loop(c * gpc, min((c + 1) * gpc, EV_G), _dedup, 0)
    d_out[n - 1].wait()
    d_out[n - 2].wait()

  @pl.when(wid < CUT)
  def _copy_a():
    _copy_loop(CP_A)

  @pl.when(wid >= CUT)
  def _copy_b():
    _copy_loop(CP_B)

  def _compact(g, off):
    v = sb[pl.ds(g * 16, 16)]
    m = v >= 0
    nodeids = row0 + g * 16 + iota
    plsc.store_compressed(nodelist.at[pl.ds(off, 16)], nodeids, mask=m)
    plsc.store_compressed(winlist.at[pl.ds(off, 16)], v, mask=m)
    return off + jnp.sum(m.astype(jnp.int32))
  cnt = lax.fori_loop(0, SB_G, _compact, jnp.int32(0))

  @pl.when(cnt > 0)
  def _pad():
    a = (cnt // 16) * 16
    padn = jnp.broadcast_to(nodelist[pl.ds(0, 16)][0], (16,))
    padw = jnp.broadcast_to(winlist[pl.ds(0, 16)][0], (16,))
    keep = iota < (cnt - a)
    nodelist[pl.ds(a, 16)] = jnp.where(keep, nodelist[pl.ds(a, 16)], padn)
    winlist[pl.ds(a, 16)] = jnp.where(keep, winlist[pl.ds(a, 16)], padw)
    for t in range(1, 9):
      nodelist[pl.ds(a + t * 16, 16)] = padn
      winlist[pl.ds(a + t * 16, 16)] = padw

  def _scatter(c, _):
    for t in range(8):
      gidx[pl.ds(t * 16, 16)] = winlist[pl.ds(c * IDX_CH + t * 16, 16)]
      nidx[pl.ds(t * 16, 16)] = nodelist[pl.ds(c * IDX_CH + t * 16, 16)]
    pltpu.async_copy(hnew_hbm.at[gidx], rowbuf, semG).wait()
    pltpu.async_copy(rowbuf, out_hbm.at[nidx], semS).wait()
    return 0
  nch = (cnt + IDX_CH - 1) // IDX_CH
  lax.fori_loop(0, nch, _scatter, 0)


def kernel(src, dst, timestamps, edge_feats, memory, last_update_t,
           time_w, time_b, W_ih, W_hh, b_ih, b_hh):
  msrc, mdst, lutg = _k1_gather(memory, last_update_t, src, dst)

  w_t = W_ih.T
  ws = w_t[:MEM_DIM]
  wd = w_t[MEM_DIM:2 * MEM_DIM]
  we = w_t[2 * MEM_DIM:2 * MEM_DIM + EDGE_DIM]
  wt = w_t[2 * MEM_DIM + EDGE_DIM:]
  hnew = _k2_call(msrc, mdst, edge_feats,
                  timestamps.reshape(B, 1), lutg.reshape(B, 1),
                  time_w.reshape(1, TEMP_DIM), time_b.reshape(1, TEMP_DIM),
                  ws, wd, we, wt, W_hh.T,
                  b_ih.reshape(1, 3 * MEM_DIM), b_hh.reshape(1, 3 * MEM_DIM))

  return _k3_scatter(memory, dst, hnew)
